# baseline (speedup 1.0000x reference)
import functools

import jax
import jax.numpy as jnp
from jax import lax
from jax.experimental import pallas as pl
from jax.experimental.pallas import tpu as pltpu
from jax.experimental.pallas import tpu_sc as plsc

N_USERS = 4000
N_ITEMS = 6000
N = N_USERS + N_ITEMS
D = 128
FPT = 4
N_LAYERS = 3
E = 320000

NC = 2
NS = 16
CE = 640
NCH = E // CE
NSTRIP = CE // 16


def _body(x0, colsg, rowsg, valsg, out,
          xa, xb, sumb,
          cbuf_0, cbuf_1, rbuf_0, rbuf_1, vbuf_0, vbuf_1,
          se_0, se_1):
    c = lax.axis_index("c")
    s = lax.axis_index("s")
    frow = (s * NC + c) * FPT

    cbuf = (cbuf_0, cbuf_1)
    rbuf = (rbuf_0, rbuf_1)
    vbuf = (vbuf_0, vbuf_1)
    se = (se_0, se_1)

    zero16 = jnp.zeros((16,), jnp.float32)
    fconsts = [jnp.full((16,), f, jnp.int32) for f in range(FPT)]

    def _fetch_edges(j, p):
        pltpu.async_copy(colsg.at[j], cbuf[p], se[p])
        pltpu.async_copy(rowsg.at[j], rbuf[p], se[p])
        pltpu.async_copy(valsg.at[j], vbuf[p], se[p])

    def _wait_edges(p):
        pltpu.make_async_copy(colsg.at[0], cbuf[p], se[p]).wait()
        pltpu.make_async_copy(rowsg.at[0], rbuf[p], se[p]).wait()
        pltpu.make_async_copy(valsg.at[0], vbuf[p], se[p]).wait()

    def _zero(ref):
        def _z(r, _):
            for f in range(FPT):
                ref[f, pl.ds(r * 16, 16)] = zero16
            return _
        lax.fori_loop(0, N // 16, _z, None)

    def _strips(p, xcur, accb):
        for st in range(NSTRIP):
            cols16 = cbuf[p][pl.ds(st * 16, 16)]
            rows16 = rbuf[p][pl.ds(st * 16, 16)]
            vals16 = vbuf[p][pl.ds(st * 16, 16)]
            for f in range(FPT):
                g = plsc.load_gather(xcur, [fconsts[f], cols16])
                plsc.addupdate_scatter(accb, [fconsts[f], rows16],
                                       g * vals16)

    def _layer_loop(xcur, accb):
        _fetch_edges(0, 0)
        _fetch_edges(1, 1)

        def _pair(g2, _):
            j = 2 * g2
            _wait_edges(0)
            _strips(0, xcur, accb)

            @pl.when(j + 2 < NCH)
            def _():
                _fetch_edges(j + 2, 0)
            _wait_edges(1)
            _strips(1, xcur, accb)

            @pl.when(j + 3 < NCH)
            def _():
                _fetch_edges(j + 3, 1)
            return _
        lax.fori_loop(0, NCH // 2, _pair, None)

    def _accum_sum(accb):
        def _a(r, _):
            for f in range(FPT):
                sumb[f, pl.ds(r * 16, 16)] = (
                    sumb[f, pl.ds(r * 16, 16)] + accb[f, pl.ds(r * 16, 16)])
            return _
        lax.fori_loop(0, N // 16, _a, None)

    pltpu.sync_copy(x0.at[pl.ds(frow, FPT), :], xa)
    _zero(sumb)
    _zero(xb)

    _layer_loop(xa, xb)
    _accum_sum(xb)
    _zero(xa)
    _layer_loop(xb, xa)
    _accum_sum(xa)
    _zero(xb)
    _layer_loop(xa, xb)
    _accum_sum(xb)

    pltpu.sync_copy(x0.at[pl.ds(frow, FPT), :], xa)
    quart = jnp.full((16,), 0.25, jnp.float32)

    def _f(r, _):
        for f in range(FPT):
            xa[f, pl.ds(r * 16, 16)] = (
                xa[f, pl.ds(r * 16, 16)] + sumb[f, pl.ds(r * 16, 16)]) * quart
        return _
    lax.fori_loop(0, N // 16, _f, None)

    pltpu.sync_copy(xa, out.at[pl.ds(frow, FPT), :])


@functools.partial(
    pl.kernel,
    out_type=jax.ShapeDtypeStruct((D, N), jnp.float32),
    mesh=plsc.VectorSubcoreMesh(core_axis_name="c", subcore_axis_name="s",
                                num_cores=NC, num_subcores=NS),
    compiler_params=pltpu.CompilerParams(use_tc_tiling_on_sc=False,
                                         needs_layout_passes=False),
    scratch_types=(
        [
            pltpu.VMEM((FPT, N), jnp.float32),
            pltpu.VMEM((FPT, N), jnp.float32),
            pltpu.VMEM((FPT, N), jnp.float32),
        ]
        + [pltpu.VMEM((CE,), jnp.int32)] * 2
        + [pltpu.VMEM((CE,), jnp.int32)] * 2
        + [pltpu.VMEM((CE,), jnp.float32)] * 2
        + [pltpu.SemaphoreType.DMA] * 2
    ),
)
def _lightgcn_sc(x0, colsg, rowsg, valsg, out, *rest):
    _body(x0, colsg, rowsg, valsg, out, *rest)


def kernel(edge_index, edge_values, user_table, item_table):
    rows = edge_index[0].astype(jnp.int32)
    cols = edge_index[1].astype(jnp.int32)
    vals = edge_values.astype(jnp.float32)

    colsg = cols.reshape(NCH, CE)
    rowsg = rows.reshape(NCH, CE)
    valsg = vals.reshape(NCH, CE)

    all_emb = jnp.concatenate([user_table, item_table], axis=0)
    x0 = all_emb.T

    out = _lightgcn_sc(x0, colsg, rowsg, valsg)
    final = out.T
    return final[:N_USERS], final[N_USERS:]


# speedup vs baseline: 4.1620x; 4.1620x over previous
import functools

import jax
import jax.numpy as jnp
from jax import lax
from jax.experimental import pallas as pl
from jax.experimental.pallas import tpu as pltpu
from jax.experimental.pallas import tpu_sc as plsc

N_USERS = 4000
N_ITEMS = 6000
N = N_USERS + N_ITEMS
D = 128
HD = D // 2
N_LAYERS = 3
E = 320000

NC = 2
NS = 16
C = 128
NCH = -(-E // (NS * C))
EP = NCH * C
EPAD = NS * EP

RPT = N // NS
RC = 125
NRC = RPT // RC

X_ROWS = (N_LAYERS + 1) * NC * N

def _body(x0, colsg, rowsg, valsg, out, xs, acc,
          cbuf0_0, cbuf0_1, rbuf_0, rbuf_1, vbuf_0, vbuf_1,
          cbuf_0, cbuf_1, srbuf_0, srbuf_1, gbuf_0, gbuf_1,
          gbufbf_0, gbufbf_1, tbbf, tmpb, sumb, zbuf,
          se_0, se_1, sg_0, sg_1, ss_0, ss_1):
    c = lax.axis_index("c")
    s = lax.axis_index("s")

    cbuf0 = (cbuf0_0, cbuf0_1)
    rbuf = (rbuf_0, rbuf_1)
    vbuf = (vbuf_0, vbuf_1)
    cbuf = (cbuf_0, cbuf_1)
    srbuf = (srbuf_0, srbuf_1)
    gbuf = (gbuf_0, gbuf_1)
    gbufbf = (gbufbf_0, gbufbf_1)
    se = (se_0, se_1)
    sg = (sg_0, sg_1)
    ss = (ss_0, ss_1)

    zero16 = jnp.zeros((16,), jnp.float32)
    zero16i = jnp.zeros((16,), jnp.int32)

    def _splat(v16, jj):
        idx = jnp.full((16, 1), jj, jnp.int32)
        dnums = lax.GatherDimensionNumbers(
            offset_dims=(), collapsed_slice_dims=(0,), start_index_map=(0,))
        return lax.gather(v16, idx, dnums, (1,),
                          mode=lax.GatherScatterMode.PROMISE_IN_BOUNDS)

    def _pack_rows(nrows):
        def _pk(r, _):
            for h in range(HD // 32):
                a = tmpb[r, pl.ds(h * 32, 16)]
                b = tmpb[r, pl.ds(h * 32 + 16, 16)]
                tbbf[r, pl.ds(h * 32, 32)] = plsc.pack(
                    a, b, format=plsc.PackFormat.INTERLEAVED)
            return _
        lax.fori_loop(0, nrows, _pk, None)

    def _fetch_edges(j, p):
        pltpu.async_copy(colsg.at[s, j], cbuf0[p], se[p])
        pltpu.async_copy(rowsg.at[s, j], rbuf[p], se[p])
        pltpu.async_copy(valsg.at[s, j], vbuf[p], se[p])

    def _wait_edges(p):
        pltpu.make_async_copy(colsg.at[s, 0], cbuf0[p], se[p]).wait()
        pltpu.make_async_copy(rowsg.at[s, 0], rbuf[p], se[p]).wait()
        pltpu.make_async_copy(valsg.at[s, 0], vbuf[p], se[p]).wait()

    def _build_cbuf(p, goff):
        offv = jnp.full((16,), goff, jnp.int32)
        for f in range(C // 16):
            cbuf[p][pl.ds(f * 16, 16)] = cbuf0[p][pl.ds(f * 16, 16)] + offv

    def _wait_scatter(p):
        pltpu.make_async_copy(gbuf[p], acc.at[srbuf[p]], ss[p]).wait()

    def _scale(p):
        for sb in range(C // 16):
            v16 = vbuf[p][pl.ds(sb * 16, 16)]
            for jj in range(16):
                valj = _splat(v16, jj)
                e = sb * 16 + jj
                for h in range(HD // 32):
                    blk = gbufbf[p][e, pl.ds(h * 32, 32)]
                    a, b = plsc.unpack(blk,
                                       format=plsc.PackFormat.INTERLEAVED)
                    gbuf[p][e, pl.ds(h * 32, 16)] = a * valj
                    gbuf[p][e, pl.ds(h * 32 + 16, 16)] = b * valj

    def _start_scatter(p):
        for f in range(C // 16):
            srbuf[p][pl.ds(f * 16, 16)] = rbuf[p][pl.ds(f * 16, 16)]
        pltpu.async_copy(gbuf[p], acc.at[srbuf[p]], ss[p], add=True)

    def _zero_zbuf(r, _):
        for f in range(HD // 16):
            zbuf[r, pl.ds(f * 16, 16)] = zero16
        return _
    lax.fori_loop(0, C, _zero_zbuf, None)

    def _zero_sumb(r, _):
        for f in range(HD // 16):
            sumb[r, pl.ds(f * 16, 16)] = zero16
        return _
    lax.fori_loop(0, RPT, _zero_sumb, None)

    def _zero_acc(k, _):
        pltpu.sync_copy(zbuf.at[pl.ds(0, RC), :],
                        acc.at[pl.ds(s * RPT + k * RC, RC), :])
        return _
    lax.fori_loop(0, NRC, _zero_acc, None)

    def _seed(k, _):
        off = c * N + s * RPT + k * RC
        pltpu.sync_copy(x0.at[pl.ds(off, RC), :], tmpb.at[pl.ds(0, RC), :])
        _pack_rows(RC)
        pltpu.sync_copy(tbbf.at[pl.ds(0, RC), :], xs.at[pl.ds(off, RC), :])
        return _
    lax.fori_loop(0, NRC, _seed, None)

    plsc.subcore_barrier()

    def _layer(l, _):
        goff = l * (NC * N) + c * N

        _fetch_edges(0, 0)
        _wait_edges(0)
        _build_cbuf(0, goff)
        pltpu.async_copy(xs.at[cbuf[0]], gbufbf[0], sg[0])
        _fetch_edges(1, 1)
        for f in range(C // 16):
            srbuf[0][pl.ds(f * 16, 16)] = zero16i
            srbuf[1][pl.ds(f * 16, 16)] = zero16i
        pltpu.async_copy(zbuf, acc.at[srbuf[0]], ss[0], add=True)
        pltpu.async_copy(zbuf, acc.at[srbuf[1]], ss[1], add=True)

        def _phase(j, p, q, last):
            if not last:
                _wait_edges(q)
                _build_cbuf(q, goff)
                pltpu.async_copy(xs.at[cbuf[q]], gbufbf[q], sg[q])
            pltpu.make_async_copy(xs.at[cbuf[p]], gbufbf[p], sg[p]).wait()
            _wait_scatter(p)
            _scale(p)
            _start_scatter(p)
            if not last:
                @pl.when(j + 2 < NCH)
                def _():
                    _fetch_edges(j + 2, p)

        def _pair(g, _):
            _phase(2 * g, 0, 1, False)
            _phase(2 * g + 1, 1, 0, False)
            return _
        lax.fori_loop(0, (NCH - 1) // 2, _pair, None)
        _phase(NCH - 1, 0, 1, True)
        _wait_scatter(0)
        _wait_scatter(1)

        plsc.subcore_barrier()

        def _drain(k, _):
            row0 = s * RPT + k * RC
            pltpu.sync_copy(acc.at[pl.ds(row0, RC), :],
                            tmpb.at[pl.ds(0, RC), :])
            woff = (l + 1) * (NC * N) + c * N + row0
            _pack_rows(RC)
            pltpu.sync_copy(tbbf.at[pl.ds(0, RC), :],
                            xs.at[pl.ds(woff, RC), :])

            def _addrow(r, _):
                for f in range(HD // 16):
                    sumb[k * RC + r, pl.ds(f * 16, 16)] = (
                        sumb[k * RC + r, pl.ds(f * 16, 16)]
                        + tmpb[r, pl.ds(f * 16, 16)])
                return _
            lax.fori_loop(0, RC, _addrow, None)

            pltpu.sync_copy(zbuf.at[pl.ds(0, RC), :],
                            acc.at[pl.ds(row0, RC), :])
            return _
        lax.fori_loop(0, NRC, _drain, None)

        plsc.subcore_barrier()
        return _
    lax.fori_loop(0, N_LAYERS, _layer, None)

    quart = jnp.full((16,), 0.25, jnp.float32)

    def _final(k, _):
        row0 = s * RPT + k * RC
        off = c * N + row0
        pltpu.sync_copy(x0.at[pl.ds(off, RC), :], tmpb.at[pl.ds(0, RC), :])

        def _outrow(r, _):
            for f in range(HD // 16):
                tmpb[r, pl.ds(f * 16, 16)] = (
                    tmpb[r, pl.ds(f * 16, 16)]
                    + sumb[k * RC + r, pl.ds(f * 16, 16)]) * quart
            return _
        lax.fori_loop(0, RC, _outrow, None)

        pltpu.sync_copy(tmpb.at[pl.ds(0, RC), :], out.at[pl.ds(off, RC), :])
        return _
    lax.fori_loop(0, NRC, _final, None)


@functools.partial(
    pl.kernel,
    out_type=(
        jax.ShapeDtypeStruct((NC * N, HD), jnp.float32),
        jax.ShapeDtypeStruct((X_ROWS, HD), jnp.bfloat16),
    ),
    mesh=plsc.VectorSubcoreMesh(core_axis_name="c", subcore_axis_name="s",
                                num_cores=NC, num_subcores=NS),
    compiler_params=pltpu.CompilerParams(use_tc_tiling_on_sc=False,
                                         needs_layout_passes=False),
    scratch_types=(
        [pltpu.VMEM_SHARED((N, HD), jnp.float32)]
        + [pltpu.VMEM((C,), jnp.int32)] * 2
        + [pltpu.VMEM((C,), jnp.int32)] * 2
        + [pltpu.VMEM((C,), jnp.float32)] * 2
        + [pltpu.VMEM((C,), jnp.int32)] * 2
        + [pltpu.VMEM((C,), jnp.int32)] * 2
        + [pltpu.VMEM((C, HD), jnp.float32)] * 2
        + [pltpu.VMEM((C, HD), jnp.bfloat16)] * 2
        + [
            pltpu.VMEM((C, HD), jnp.bfloat16),
            pltpu.VMEM((C, HD), jnp.float32),
            pltpu.VMEM((RPT, HD), jnp.float32),
            pltpu.VMEM((C, HD), jnp.float32),
        ]
        + [pltpu.SemaphoreType.DMA] * 6
    ),
)
def _lightgcn_sc(x0, colsg, rowsg, valsg, out, xs, *rest):
    _body(x0, colsg, rowsg, valsg, out, xs, *rest)


def kernel(edge_index, edge_values, user_table, item_table):
    rows = edge_index[0].astype(jnp.int32)
    cols = edge_index[1].astype(jnp.int32)
    vals = edge_values.astype(jnp.float32)

    pad = EPAD - E
    rows = jnp.concatenate([rows, jnp.zeros((pad,), jnp.int32)])
    cols = jnp.concatenate([cols, jnp.zeros((pad,), jnp.int32)])
    vals = jnp.concatenate([vals, jnp.zeros((pad,), jnp.float32)])

    colsg = cols.reshape(NS, NCH, C)
    rowsg = rows.reshape(NS, NCH, C)
    valsg = vals.reshape(NS, NCH, C)

    all_emb = jnp.concatenate([user_table, item_table], axis=0)
    x0 = jnp.concatenate([all_emb[:, :HD], all_emb[:, HD:]], axis=0)

    out, _ = _lightgcn_sc(x0, colsg, rowsg, valsg)
    final = jnp.concatenate([out[:N], out[N:]], axis=1)
    return final[:N_USERS], final[N_USERS:]
